# Initial kernel scaffold; baseline (speedup 1.0000x reference)
import jax
import jax.numpy as jnp
from jax.experimental import pallas as pl


def kernel(history_data, retrieval_keys, retrieval_values, W_temp, b_temp, spatial_encoder, W_enc1, b_enc1, W_enc2, b_enc2, W_hq, b_hq, W_qr, b_qr, Wq_a, bq_a, Wk_a, bk_a, Wv_a, bv_a, Wo_a, bo_a, W1, b1, W2, b2):
    raise NotImplementedError("write your pallas kernel here")


# speedup vs baseline: 2.3315x; 2.3315x over previous
import functools

import jax
import jax.numpy as jnp
from jax import lax
from jax.experimental import pallas as pl
from jax.experimental.pallas import tpu as pltpu
from jax.experimental.pallas import tpu_sc as plsc

_B = 64
_L = 12
_N = 207
_C = 3
_TD = 64
_SD = 64
_FD = _TD + _SD
_QD = 32
_RD = 64
_K = 10000
_TOPK = 3
_H = 12
_HID = 256
_NH = 4
_DH = _RD // _NH

_Q = _B * _N
_TQ = 256
_QPAD = 13312
_GRID = _QPAD // _TQ
_KB = 2048
_KPAD = 10240
_NKB = _KPAD // _KB

_NEG = -3.0e38
_BIGI = 1 << 30

_SC_NC = 2
_SC_NS = 16
_SC_NW = _SC_NC * _SC_NS
_GB = _TOPK * _QPAD
_GPW = _GB // _SC_NW
_GCH = 104
_GNC = _GPW // _GCH
_VD = 128


def _enc_topk_body(x_ref, sp_ref, keys_ref, wt_ref, bt_ref, w1_ref, b1_ref,
                   w2_ref, b2_ref, whq_ref, bhq_ref, wqr_ref, bqr_ref,
                   q_out, qr_out, idx_out):
    x = x_ref[...]
    temp = lax.dot_general(x, wt_ref[...], (((1,), (1,)), ((), ())),
                           preferred_element_type=jnp.float32) + bt_ref[...]
    hidden = jnp.concatenate([temp, sp_ref[...]], axis=1)
    h1 = jnp.maximum(
        lax.dot_general(hidden, w1_ref[...], (((1,), (1,)), ((), ())),
                        preferred_element_type=jnp.float32) + b1_ref[...], 0.0)
    h2 = jnp.maximum(
        lax.dot_general(h1, w2_ref[...], (((1,), (1,)), ((), ())),
                        preferred_element_type=jnp.float32) + b2_ref[...], 0.0)
    query = lax.dot_general(h2, whq_ref[...], (((1,), (1,)), ((), ())),
                            preferred_element_type=jnp.float32) + bhq_ref[...]
    qret = lax.dot_general(query, wqr_ref[...], (((1,), (1,)), ((), ())),
                           preferred_element_type=jnp.float32) + bqr_ref[...]
    q_out[...] = query
    qr_out[...] = qret

    cand_v = []
    cand_i = []
    for kb in range(_NKB):
        kblk = keys_ref[kb * _KB:(kb + 1) * _KB, :]
        s = lax.dot_general(qret, kblk, (((1,), (1,)), ((), ())),
                            preferred_element_type=jnp.float32)
        iot = lax.broadcasted_iota(jnp.int32, (_TQ, _KB), 1) + (kb * _KB)
        if (kb + 1) * _KB > _K:
            s = jnp.where(iot < _K, s, _NEG)
        for t in range(_TOPK):
            m = jnp.max(s, axis=1, keepdims=True)
            im = jnp.min(jnp.where(s >= m, iot, _BIGI), axis=1,
                         keepdims=True)
            cand_v.append(m)
            cand_i.append(im)
            if t < _TOPK - 1:
                s = jnp.where(iot == im, _NEG, s)

    cv = jnp.concatenate(cand_v, axis=1)
    ci = jnp.concatenate(cand_i, axis=1)
    picks = []
    for t in range(_TOPK):
        m = jnp.max(cv, axis=1, keepdims=True)
        eq = cv >= m
        im = jnp.min(jnp.where(eq, ci, _BIGI), axis=1, keepdims=True)
        picks.append(im)
        if t < _TOPK - 1:
            cv = jnp.where(eq & (ci == im), _NEG, cv)
    idx_out[...] = jnp.concatenate(
        picks + [picks[0]] * (8 - _TOPK), axis=1)


def _attn_mlp_body(qr_ref, q_ref, rv_ref, wq_ref, bq_ref, wk_ref, bk_ref,
                   wv_ref, bv_ref, wo_ref, bo_ref, w1a_ref, w1b_ref, b1_ref,
                   w2_ref, b2_ref, out_ref):
    qret = qr_ref[...]
    query = q_ref[...]
    qh = lax.dot_general(qret, wq_ref[...], (((1,), (1,)), ((), ())),
                         preferred_element_type=jnp.float32) + bq_ref[...]
    gi = lax.broadcasted_iota(jnp.int32, (_RD, _NH), 0) // _DH
    gj = lax.broadcasted_iota(jnp.int32, (_RD, _NH), 1)
    gsum = (gi == gj).astype(jnp.float32)
    scale = 1.0 / (_DH ** 0.5)

    logits = []
    vhs = []
    for t in range(_TOPK):
        rv = rv_ref[t]
        kh = lax.dot_general(rv, wk_ref[...], (((1,), (1,)), ((), ())),
                             preferred_element_type=jnp.float32) + bk_ref[...]
        vh = lax.dot_general(rv, wv_ref[...], (((1,), (1,)), ((), ())),
                             preferred_element_type=jnp.float32) + bv_ref[...]
        vhs.append(vh)
        logits.append(
            lax.dot_general(qh * kh, gsum, (((1,), (0,)), ((), ())),
                            preferred_element_type=jnp.float32) * scale)

    m = jnp.maximum(jnp.maximum(logits[0], logits[1]), logits[2])
    es = [jnp.exp(l - m) for l in logits]
    z = es[0] + es[1] + es[2]
    attn = jnp.zeros((_TQ, _RD), jnp.float32)
    for t in range(_TOPK):
        w = es[t] / z
        wex = lax.dot_general(w, gsum, (((1,), (1,)), ((), ())),
                              preferred_element_type=jnp.float32)
        attn = attn + wex * vhs[t]
    ao = lax.dot_general(attn, wo_ref[...], (((1,), (1,)), ((), ())),
                         preferred_element_type=jnp.float32) + bo_ref[...]
    h = jnp.maximum(
        lax.dot_general(ao, w1a_ref[...], (((1,), (1,)), ((), ())),
                        preferred_element_type=jnp.float32)
        + lax.dot_general(query, w1b_ref[...], (((1,), (1,)), ((), ())),
                          preferred_element_type=jnp.float32)
        + b1_ref[...], 0.0)
    out_ref[...] = lax.dot_general(h, w2_ref[...], (((1,), (1,)), ((), ())),
                                   preferred_element_type=jnp.float32) + b2_ref[...]


_sc_gather_built = None


def _build_sc_gather():
    @functools.partial(
        pl.kernel,
        mesh=plsc.VectorSubcoreMesh(core_axis_name="c", subcore_axis_name="s"),
        out_type=jax.ShapeDtypeStruct((_SC_NW * _GNC, _GCH, _VD), jnp.float32),
        scratch_types=[
            pltpu.VMEM((_GNC, _GCH), jnp.int32),
            pltpu.VMEM((_GCH, _VD), jnp.float32),
            pltpu.VMEM((_GCH, _VD), jnp.float32),
            pltpu.SemaphoreType.DMA,
            pltpu.SemaphoreType.DMA,
        ],
    )
    def gather_k(values_hbm, idx_hbm, out_hbm, idx_s, rows_a, rows_b, sem_a, sem_b):
        wid = lax.axis_index("s") * _SC_NC + lax.axis_index("c")
        pltpu.sync_copy(idx_hbm.at[wid], idx_s)
        bufs = (rows_a, rows_b)
        sems = (sem_a, sem_b)
        cps = [None, None]
        for c in range(_GNC):
            p = c & 1
            cps[p] = pltpu.async_copy(values_hbm.at[idx_s.at[c]], bufs[p], sems[p])
            if c > 0:
                cps[1 - p].wait()
                pltpu.sync_copy(bufs[1 - p], out_hbm.at[wid * _GNC + c - 1])
        cps[(_GNC - 1) & 1].wait()
        pltpu.sync_copy(bufs[(_GNC - 1) & 1], out_hbm.at[wid * _GNC + _GNC - 1])

    return gather_k


def _sc_gather(values_p, idx):
    global _sc_gather_built
    if _sc_gather_built is None:
        _sc_gather_built = _build_sc_gather()
    idx3 = idx.reshape(_SC_NW, _GNC, _GCH)
    return _sc_gather_built(values_p, idx3).reshape(_GB, _VD)


def kernel(history_data, retrieval_keys, retrieval_values, W_temp, b_temp,
           spatial_encoder, W_enc1, b_enc1, W_enc2, b_enc2, W_hq, b_hq,
           W_qr, b_qr, Wq_a, bq_a, Wk_a, bk_a, Wv_a, bv_a, Wo_a, bo_a,
           W1, b1, W2, b2):
    f32 = jnp.float32
    x = history_data.transpose(0, 2, 1, 3).reshape(_Q, _L * _C)
    x = jnp.pad(x, ((0, _QPAD - _Q), (0, 0)))
    sp = jnp.broadcast_to(spatial_encoder[None], (_B, _N, _SD)).reshape(_Q, _SD)
    sp = jnp.pad(sp, ((0, _QPAD - _Q), (0, 0)))
    keys_p = jnp.pad(retrieval_keys, ((0, _KPAD - _K), (0, 0)))

    row = lambda b: b.reshape(1, -1).astype(f32)

    grid1 = pl.GridSpec(
        grid=(_GRID,),
        in_specs=[
            pl.BlockSpec((_TQ, _L * _C), lambda i: (i, 0)),
            pl.BlockSpec((_TQ, _SD), lambda i: (i, 0)),
            pl.BlockSpec((_KPAD, _RD), lambda i: (0, 0)),
            pl.BlockSpec((_TD, _L * _C), lambda i: (0, 0)),
            pl.BlockSpec((1, _TD), lambda i: (0, 0)),
            pl.BlockSpec((_FD, _FD), lambda i: (0, 0)),
            pl.BlockSpec((1, _FD), lambda i: (0, 0)),
            pl.BlockSpec((_FD, _FD), lambda i: (0, 0)),
            pl.BlockSpec((1, _FD), lambda i: (0, 0)),
            pl.BlockSpec((_QD, _FD), lambda i: (0, 0)),
            pl.BlockSpec((1, _QD), lambda i: (0, 0)),
            pl.BlockSpec((_RD, _QD), lambda i: (0, 0)),
            pl.BlockSpec((1, _RD), lambda i: (0, 0)),
        ],
        out_specs=[
            pl.BlockSpec((_TQ, _QD), lambda i: (i, 0)),
            pl.BlockSpec((_TQ, _RD), lambda i: (i, 0)),
            pl.BlockSpec((_TQ, 8), lambda i: (i, 0)),
        ],
    )
    query, qret, idx8 = pl.pallas_call(
        _enc_topk_body,
        grid_spec=grid1,
        out_shape=[
            jax.ShapeDtypeStruct((_QPAD, _QD), f32),
            jax.ShapeDtypeStruct((_QPAD, _RD), f32),
            jax.ShapeDtypeStruct((_QPAD, 8), jnp.int32),
        ],
    )(x, sp, keys_p, W_temp, row(b_temp), W_enc1, row(b_enc1), W_enc2,
      row(b_enc2), W_hq, row(b_hq), W_qr, row(b_qr))

    idx_flat = idx8[:, :_TOPK].T.reshape(_GB)
    values_p = jnp.pad(retrieval_values, ((0, 0), (0, _VD - _RD)))
    gathered = _sc_gather(values_p, idx_flat)
    rv = gathered[:, :_RD].reshape(_TOPK, _QPAD, _RD)

    grid2 = pl.GridSpec(
        grid=(_GRID,),
        in_specs=[
            pl.BlockSpec((_TQ, _RD), lambda i: (i, 0)),
            pl.BlockSpec((_TQ, _QD), lambda i: (i, 0)),
            pl.BlockSpec((_TOPK, _TQ, _RD), lambda i: (0, i, 0)),
            pl.BlockSpec((_RD, _RD), lambda i: (0, 0)),
            pl.BlockSpec((1, _RD), lambda i: (0, 0)),
            pl.BlockSpec((_RD, _RD), lambda i: (0, 0)),
            pl.BlockSpec((1, _RD), lambda i: (0, 0)),
            pl.BlockSpec((_RD, _RD), lambda i: (0, 0)),
            pl.BlockSpec((1, _RD), lambda i: (0, 0)),
            pl.BlockSpec((_RD, _RD), lambda i: (0, 0)),
            pl.BlockSpec((1, _RD), lambda i: (0, 0)),
            pl.BlockSpec((_HID, _RD), lambda i: (0, 0)),
            pl.BlockSpec((_HID, _QD), lambda i: (0, 0)),
            pl.BlockSpec((1, _HID), lambda i: (0, 0)),
            pl.BlockSpec((_H, _HID), lambda i: (0, 0)),
            pl.BlockSpec((1, _H), lambda i: (0, 0)),
        ],
        out_specs=pl.BlockSpec((_TQ, _H), lambda i: (i, 0)),
    )
    out = pl.pallas_call(
        _attn_mlp_body,
        grid_spec=grid2,
        out_shape=jax.ShapeDtypeStruct((_QPAD, _H), f32),
    )(qret, query, rv, Wq_a, row(bq_a), Wk_a, row(bk_a), Wv_a, row(bv_a),
      Wo_a, row(bo_a), W1[:, :_RD], W1[:, _RD:], row(b1), W2, row(b2))

    return out[:_Q].reshape(_B, _N, _H, 1).transpose(0, 2, 1, 3)
